# baseline (speedup 1.0000x reference)
import functools

import jax
import jax.numpy as jnp
from jax import lax
from jax.experimental import pallas as pl
from jax.experimental.pallas import tpu as pltpu
from jax.experimental.pallas import tpu_sc as plsc

BATCH = 4096
SEQ_LEN = 200
VOCAB = 6
DIM = 64

N_ROWS = BATCH * SEQ_LEN
NC, NS = 2, 16
NW = NC * NS
ROWS_PER_W = N_ROWS // NW
IDX_MINOR = 128
GROUPS_PER_W = ROWS_PER_W // IDX_MINOR
CH = 8
CHUNK_ROWS = CH * IDX_MINOR
N_CHUNKS = GROUPS_PER_W // CH


def _mesh():
    return plsc.VectorSubcoreMesh(
        core_axis_name="c", subcore_axis_name="s",
        num_cores=NC, num_subcores=NS)


@functools.partial(
    pl.kernel,
    out_type=jax.ShapeDtypeStruct((N_ROWS, DIM), jnp.float32),
    mesh=_mesh(),
    compiler_params=pltpu.CompilerParams(use_tc_tiling_on_sc=False),
    scratch_types=[
        pltpu.VMEM((CH, IDX_MINOR), jnp.int32),
        pltpu.VMEM((CHUNK_ROWS, DIM), jnp.float32),
        pltpu.SemaphoreType.DMA,
    ],
)
def _gather_kernel(table_hbm, idx_hbm, out_hbm, idx_v, rows_v, sem):
    wid = lax.axis_index("s") * NC + lax.axis_index("c")
    g_base = wid * GROUPS_PER_W
    r_base = wid * ROWS_PER_W

    def chunk(t, carry):
        g0 = g_base + t * CH
        r0 = r_base + t * CHUNK_ROWS
        pltpu.sync_copy(idx_hbm.at[pl.ds(g0, CH)], idx_v)
        copies = [
            pltpu.async_copy(
                table_hbm.at[idx_v.at[j]],
                rows_v.at[pl.ds(j * IDX_MINOR, IDX_MINOR)],
                sem)
            for j in range(CH)
        ]
        for c in copies:
            c.wait()
        pltpu.sync_copy(rows_v, out_hbm.at[pl.ds(r0, CHUNK_ROWS)])
        return carry

    lax.fori_loop(0, N_CHUNKS, chunk, 0)


def kernel(indices, table):
    idx = indices.reshape(N_ROWS // IDX_MINOR, IDX_MINOR).astype(jnp.int32)
    return _gather_kernel(table, idx)


# speedup vs baseline: 6.2859x; 6.2859x over previous
import functools

import jax
import jax.numpy as jnp
from jax import lax
from jax.experimental import pallas as pl
from jax.experimental.pallas import tpu as pltpu
from jax.experimental.pallas import tpu_sc as plsc

BATCH = 4096
SEQ_LEN = 200
VOCAB = 6
DIM = 64

N_ROWS = BATCH * SEQ_LEN
NC, NS = 2, 16
NW = NC * NS
ROWS_PER_W = N_ROWS // NW
CHUNK_ROWS = 1024
N_CHUNKS = ROWS_PER_W // CHUNK_ROWS
L = 16


def _mesh():
    return plsc.VectorSubcoreMesh(
        core_axis_name="c", subcore_axis_name="s",
        num_cores=NC, num_subcores=NS)


@functools.partial(
    pl.kernel,
    out_type=jax.ShapeDtypeStruct((N_ROWS * DIM,), jnp.float32),
    mesh=_mesh(),
    compiler_params=pltpu.CompilerParams(use_tc_tiling_on_sc=False),
    scratch_types=[
        pltpu.VMEM((VOCAB * DIM,), jnp.float32),
        pltpu.VMEM((CHUNK_ROWS,), jnp.int32),
        pltpu.VMEM((CHUNK_ROWS * DIM,), jnp.float32),
        pltpu.SemaphoreType.DMA,
    ],
)
def _expand_kernel(table_hbm, idx_hbm, out_hbm, table_v, idx_v, rows_v, sem):
    wid = lax.axis_index("s") * NC + lax.axis_index("c")
    r_base = wid * ROWS_PER_W

    pltpu.sync_copy(table_hbm, table_v)

    def chunk(t, carry):
        r0 = r_base + t * CHUNK_ROWS
        pltpu.sync_copy(idx_hbm.at[pl.ds(r0, CHUNK_ROWS)], idx_v)

        def expand(g, c2):
            offs = idx_v[pl.ds(g * L, L)] * DIM
            d0 = g * L * DIM
            for k in range(L):
                o = offs[k]
                d = d0 + k * DIM
                for c in range(DIM // L):
                    rows_v[pl.ds(d + c * L, L)] = table_v[pl.ds(o + c * L, L)]
            return c2

        lax.fori_loop(0, CHUNK_ROWS // L, expand, 0)
        pltpu.sync_copy(rows_v, out_hbm.at[pl.ds(r0 * DIM, CHUNK_ROWS * DIM)])
        return carry

    lax.fori_loop(0, N_CHUNKS, chunk, 0)


def kernel(indices, table):
    idx = indices.reshape(N_ROWS).astype(jnp.int32)
    flat = _expand_kernel(table.reshape(VOCAB * DIM), idx)
    return flat.reshape(N_ROWS, DIM)
